# Initial kernel scaffold; baseline (speedup 1.0000x reference)
import jax
import jax.numpy as jnp
from jax.experimental import pallas as pl


def kernel(x, edge_index, W, b, Wl, bl):
    raise NotImplementedError("write your pallas kernel here")


# speedup vs baseline: 145.0876x; 145.0876x over previous
import functools

import jax
import jax.numpy as jnp
from jax import lax
from jax.experimental import pallas as pl
from jax.experimental.pallas import tpu as pltpu
from jax.experimental.pallas import tpu_sc as plsc

N = 100000
NP = 102400
TRASH = N
E = 6400000
NC, NS = 2, 16
NW = NC * NS
ROW = 128
RPW = 1568
CH = 7
ITERS = RPW // CH
ROWS_TOT = NW * RPW
E_PAD = ROWS_TOT * ROW
NSL = NP // NS
NV = NSL // 16
NV8 = NSL * 8 // 16
NPASS = 16
PR = NSL // NPASS
PV = PR * 8 // 16

_mesh = plsc.VectorSubcoreMesh(core_axis_name="c", subcore_axis_name="s")
_sc_params = pltpu.CompilerParams(use_tc_tiling_on_sc=False,
                                  needs_layout_passes=False)


def _newton_rsqrt(d):
    u = plsc.bitcast(d, jnp.int32)
    u = 0x5F3759DF - lax.shift_right_logical(u, 1)
    g = plsc.bitcast(u, jnp.float32)
    for _ in range(3):
        g = g * (1.5 - 0.5 * d * g * g)
    return g


@functools.partial(
    pl.kernel,
    out_type=jax.ShapeDtypeStruct((NC, NP), jnp.float32),
    mesh=_mesh,
    compiler_params=_sc_params,
    scratch_types=[
        pltpu.VMEM_SHARED((NP,), jnp.float32),
    ],
)
def _deg_kernel(dst_hbm, deg_out, deg_sh):
    cid = lax.axis_index("c")
    sid = lax.axis_index("s")
    wid = cid * NS + sid
    row0 = wid * RPW

    def scoped(zb, idx_a, idx_b, ones_v, isem_a, isem_b, sem_a, sem_b):
        zv = jnp.zeros((16,), jnp.float32)

        def zinit(v, carry):
            zb[pl.ds(v * 16, 16)] = zv
            return carry

        lax.fori_loop(0, NV, zinit, 0)
        for i in range(8):
            ones_v[pl.ds(i * 16, 16)] = zv + 1.0
        pltpu.sync_copy(zb, deg_sh.at[pl.ds(sid * NSL, NSL)])
        plsc.subcore_barrier()

        pltpu.sync_copy(dst_hbm.at[pl.ds(row0, CH)], idx_a)

        def body(j, carry):
            r = row0 + j * 2 * CH
            ib = pltpu.async_copy(dst_hbm.at[pl.ds(r + CH, CH)], idx_b, isem_b)
            sa = [
                pltpu.async_copy(ones_v, deg_sh.at[idx_a.at[i]], sem_a, add=True)
                for i in range(CH)
            ]
            ib.wait()
            sb = [
                pltpu.async_copy(ones_v, deg_sh.at[idx_b.at[i]], sem_b, add=True)
                for i in range(CH)
            ]
            for d in sa:
                d.wait()
            nxt = row0 + ((2 * j + 2) % ITERS) * CH
            ia = pltpu.async_copy(dst_hbm.at[pl.ds(nxt, CH)], idx_a, isem_a)
            for d in sb:
                d.wait()
            ia.wait()
            return carry

        lax.fori_loop(0, ITERS // 2, body, 0)
        plsc.subcore_barrier()
        pltpu.sync_copy(deg_sh.at[pl.ds(sid * NSL, NSL)],
                        deg_out.at[cid, pl.ds(sid * NSL, NSL)])

    pl.run_scoped(scoped,
                  pltpu.VMEM((NSL,), jnp.float32),
                  pltpu.VMEM((CH, ROW), jnp.int32),
                  pltpu.VMEM((CH, ROW), jnp.int32),
                  pltpu.VMEM((ROW,), jnp.float32),
                  pltpu.SemaphoreType.DMA,
                  pltpu.SemaphoreType.DMA,
                  pltpu.SemaphoreType.DMA,
                  pltpu.SemaphoreType.DMA)


@functools.partial(
    pl.kernel,
    out_type=jax.ShapeDtypeStruct((NC, NP, 8), jnp.float32),
    mesh=_mesh,
    compiler_params=_sc_params,
    scratch_types=[
        pltpu.VMEM_SHARED((NP, 8), jnp.float32),
        pltpu.VMEM_SHARED((NP, 8), jnp.float32),
    ],
)
def _agg_kernel(src_hbm, dst_hbm, x_hbm, deg_hbm, agg_out, y_sh, s_sh):
    cid = lax.axis_index("c")
    sid = lax.axis_index("s")
    wid = cid * NS + sid
    row0 = wid * RPW
    sl1 = pl.ds(sid * NSL, NSL)
    sl2 = pl.ds(sid * NSL, NSL)
    iota = lax.iota(jnp.int32, 16)
    rbase = lax.shift_right_logical(iota, 3)
    cols = jnp.bitwise_and(iota, 7)

    def scoped(xb, dinvb, d0b, d1b, src_a, dst_a, src_b, dst_b, rows_a, rows_b,
               isem_a, isem_b, gsem_a, gsem_b, ssem_a, ssem_b):
        def dinv_pass(h):
            slh1 = pl.ds(sid * NSL + h * PR, PR)
            pltpu.sync_copy(deg_hbm.at[0, slh1], d0b)
            pltpu.sync_copy(deg_hbm.at[1, slh1], d1b)

            def dloop(v, carry):
                ds = pl.ds(v * 16, 16)
                dinvb[ds] = _newton_rsqrt(d0b[ds] + d1b[ds] + 1.0)
                return carry

            lax.fori_loop(0, PR // 16, dloop, 0)

        def stage1(h, carry):
            slh = pl.ds(sid * NSL + h * PR, PR)
            dinv_pass(h)
            pltpu.sync_copy(x_hbm.at[slh], xb)

            def yloop(j, carry):
                for k in range(4):
                    v = j * 4 + k
                    rows = rbase + 2 * v
                    xv = plsc.load_gather(xb, [rows, cols])
                    dv = plsc.load_gather(dinvb, [rows])
                    plsc.store_scatter(xb, [rows, cols], dv * xv)
                return carry

            lax.fori_loop(0, PV // 4, yloop, 0)
            pltpu.sync_copy(xb, y_sh.at[slh])

            @pl.when(cid == 1)
            def _():
                zv = jnp.zeros((16,), jnp.float32)

                def zloop(j, carry):
                    for k in range(4):
                        v = j * 4 + k
                        plsc.store_scatter(xb, [rbase + 2 * v, cols], zv)
                    return carry

                lax.fori_loop(0, PV // 4, zloop, 0)

            pltpu.sync_copy(xb, s_sh.at[slh])
            return carry

        lax.fori_loop(0, NPASS, stage1, 0)
        plsc.subcore_barrier()

        pltpu.sync_copy(src_hbm.at[pl.ds(row0, CH)], src_a)
        pltpu.sync_copy(dst_hbm.at[pl.ds(row0, CH)], dst_a)

        def body(j, carry):
            r = row0 + j * 2 * CH
            ib1 = pltpu.async_copy(src_hbm.at[pl.ds(r + CH, CH)], src_b, isem_b)
            ib2 = pltpu.async_copy(dst_hbm.at[pl.ds(r + CH, CH)], dst_b, isem_b)
            ga = [
                pltpu.async_copy(y_sh.at[src_a.at[i]], rows_a.at[i], gsem_a)
                for i in range(CH)
            ]
            for d in ga:
                d.wait()
            sa = [
                pltpu.async_copy(rows_a.at[i], s_sh.at[dst_a.at[i]], ssem_a,
                                 add=True)
                for i in range(CH)
            ]
            ib1.wait()
            ib2.wait()
            gb = [
                pltpu.async_copy(y_sh.at[src_b.at[i]], rows_b.at[i], gsem_b)
                for i in range(CH)
            ]
            for d in gb:
                d.wait()
            sb = [
                pltpu.async_copy(rows_b.at[i], s_sh.at[dst_b.at[i]], ssem_b,
                                 add=True)
                for i in range(CH)
            ]
            for d in sa:
                d.wait()
            nxt = row0 + ((2 * j + 2) % ITERS) * CH
            ia1 = pltpu.async_copy(src_hbm.at[pl.ds(nxt, CH)], src_a, isem_a)
            ia2 = pltpu.async_copy(dst_hbm.at[pl.ds(nxt, CH)], dst_a, isem_a)
            for d in sb:
                d.wait()
            ia1.wait()
            ia2.wait()
            return carry

        lax.fori_loop(0, ITERS // 2, body, 0)
        plsc.subcore_barrier()

        def stage3(h, carry):
            slh = pl.ds(sid * NSL + h * PR, PR)
            dinv_pass(h)
            pltpu.sync_copy(s_sh.at[slh], xb)

            def aloop(j, carry):
                for k in range(4):
                    v = j * 4 + k
                    rows = rbase + 2 * v
                    sv = plsc.load_gather(xb, [rows, cols])
                    dv = plsc.load_gather(dinvb, [rows])
                    plsc.store_scatter(xb, [rows, cols], dv * sv)
                return carry

            lax.fori_loop(0, PV // 4, aloop, 0)
            pltpu.sync_copy(xb, agg_out.at[cid, slh])
            return carry

        lax.fori_loop(0, NPASS, stage3, 0)

    pl.run_scoped(scoped,
                  pltpu.VMEM((PR, 8), jnp.float32),
                  pltpu.VMEM((PR,), jnp.float32),
                  pltpu.VMEM((PR,), jnp.float32),
                  pltpu.VMEM((PR,), jnp.float32),
                  pltpu.VMEM((CH, ROW), jnp.int32),
                  pltpu.VMEM((CH, ROW), jnp.int32),
                  pltpu.VMEM((CH, ROW), jnp.int32),
                  pltpu.VMEM((CH, ROW), jnp.int32),
                  pltpu.VMEM((CH, ROW, 8), jnp.float32),
                  pltpu.VMEM((CH, ROW, 8), jnp.float32),
                  pltpu.SemaphoreType.DMA,
                  pltpu.SemaphoreType.DMA,
                  pltpu.SemaphoreType.DMA,
                  pltpu.SemaphoreType.DMA,
                  pltpu.SemaphoreType.DMA,
                  pltpu.SemaphoreType.DMA)


def _hidden_body(a_ref, w_ref, b_ref, hid_ref):
    agg = a_ref[0] + a_ref[1]
    hid_ref[...] = (
        jnp.dot(agg, w_ref[...], preferred_element_type=jnp.float32) + b_ref[...]
    )


def _hidden(agg_parts, W, b2):
    blk = 2048
    grid = (NP // blk,)
    return pl.pallas_call(
        _hidden_body,
        grid=grid,
        in_specs=[
            pl.BlockSpec((NC, blk, 8), lambda i: (0, i, 0)),
            pl.BlockSpec((8, 16), lambda i: (0, 0)),
            pl.BlockSpec((1, 16), lambda i: (0, 0)),
        ],
        out_specs=pl.BlockSpec((blk, 16), lambda i: (i, 0)),
        out_shape=jax.ShapeDtypeStruct((NP, 16), jnp.float32),
    )(agg_parts, W, b2)


def _lin_body(h_ref, wl_ref, bl_ref, out_ref):
    h = jnp.maximum(h_ref[...], 0.0)
    out_ref[...] = (
        jnp.dot(h, wl_ref[...], preferred_element_type=jnp.float32) + bl_ref[...]
    )


def _linear(h2, Wl, bl2):
    m = h2.shape[0]
    blk = 2000
    grid = (m // blk,)
    return pl.pallas_call(
        _lin_body,
        grid=grid,
        in_specs=[
            pl.BlockSpec((blk, 80), lambda i: (i, 0)),
            pl.BlockSpec((80, 445), lambda i: (0, 0)),
            pl.BlockSpec((1, 445), lambda i: (0, 0)),
        ],
        out_specs=pl.BlockSpec((blk, 445), lambda i: (i, 0)),
        out_shape=jax.ShapeDtypeStruct((m, 445), jnp.float32),
    )(h2, Wl, bl2)


def kernel(x, edge_index, W, b, Wl, bl):
    src = edge_index[0]
    dst = edge_index[1]
    pad = jnp.full((E_PAD - E,), TRASH, dtype=jnp.int32)
    src_p = jnp.concatenate([src, pad]).reshape(ROWS_TOT, ROW)
    dst_p = jnp.concatenate([dst, pad]).reshape(ROWS_TOT, ROW)
    x_pad = jnp.pad(x, ((0, NP - N), (0, 0)))

    deg_parts = _deg_kernel(dst_p)
    agg_parts = _agg_kernel(src_p, dst_p, x_pad, deg_parts)
    hidden_pad = _hidden(agg_parts, W, b.reshape(1, 16))
    hidden = hidden_pad[:N]
    h2 = hidden.reshape(N // 5, 80)
    out = _linear(h2, Wl, bl.reshape(1, 445)).reshape(N // 5, 89, 5)
    return (out, hidden)


# speedup vs baseline: 148.8880x; 1.0262x over previous
import functools

import jax
import jax.numpy as jnp
from jax import lax
from jax.experimental import pallas as pl
from jax.experimental.pallas import tpu as pltpu
from jax.experimental.pallas import tpu_sc as plsc

N = 100000
NP = 102400
TRASH = N
E = 6400000
NC, NS = 2, 16
NW = NC * NS
ROW = 128
RPW = 1568
CH = 7
ITERS = RPW // CH
ROWS_TOT = NW * RPW
E_PAD = ROWS_TOT * ROW
NSL = NP // NS
NV = NSL // 16
NV8 = NSL * 8 // 16
NPASS = 16
PR = NSL // NPASS
PV = PR * 8 // 16

_mesh = plsc.VectorSubcoreMesh(core_axis_name="c", subcore_axis_name="s")
_sc_params = pltpu.CompilerParams(use_tc_tiling_on_sc=False,
                                  needs_layout_passes=False)


def _newton_rsqrt(d):
    u = plsc.bitcast(d, jnp.int32)
    u = 0x5F3759DF - lax.shift_right_logical(u, 1)
    g = plsc.bitcast(u, jnp.float32)
    for _ in range(3):
        g = g * (1.5 - 0.5 * d * g * g)
    return g


@functools.partial(
    pl.kernel,
    out_type=jax.ShapeDtypeStruct((NC, NP), jnp.float32),
    mesh=_mesh,
    compiler_params=_sc_params,
    scratch_types=[
        pltpu.VMEM_SHARED((NP,), jnp.float32),
    ],
)
def _deg_kernel(dst_hbm, deg_out, deg_sh):
    cid = lax.axis_index("c")
    sid = lax.axis_index("s")
    wid = cid * NS + sid
    row0 = wid * RPW

    def scoped(zb, idx_a, idx_b, ones_v, isem_a, isem_b, sem_a, sem_b):
        zv = jnp.zeros((16,), jnp.float32)

        def zinit(v, carry):
            zb[pl.ds(v * 16, 16)] = zv
            return carry

        lax.fori_loop(0, NV, zinit, 0)
        for i in range(8):
            ones_v[pl.ds(i * 16, 16)] = zv + 1.0
        pltpu.sync_copy(zb, deg_sh.at[pl.ds(sid * NSL, NSL)])
        plsc.subcore_barrier()

        pltpu.sync_copy(dst_hbm.at[pl.ds(row0, CH)], idx_a)

        def body(j, carry):
            r = row0 + j * 2 * CH
            ib = pltpu.async_copy(dst_hbm.at[pl.ds(r + CH, CH)], idx_b, isem_b)
            sa = [
                pltpu.async_copy(ones_v, deg_sh.at[idx_a.at[i]], sem_a, add=True)
                for i in range(CH)
            ]
            ib.wait()
            sb = [
                pltpu.async_copy(ones_v, deg_sh.at[idx_b.at[i]], sem_b, add=True)
                for i in range(CH)
            ]
            for d in sa:
                d.wait()
            nxt = row0 + ((2 * j + 2) % ITERS) * CH
            ia = pltpu.async_copy(dst_hbm.at[pl.ds(nxt, CH)], idx_a, isem_a)
            for d in sb:
                d.wait()
            ia.wait()
            return carry

        lax.fori_loop(0, ITERS // 2, body, 0)
        plsc.subcore_barrier()
        pltpu.sync_copy(deg_sh.at[pl.ds(sid * NSL, NSL)],
                        deg_out.at[cid, pl.ds(sid * NSL, NSL)])

    pl.run_scoped(scoped,
                  pltpu.VMEM((NSL,), jnp.float32),
                  pltpu.VMEM((CH, ROW), jnp.int32),
                  pltpu.VMEM((CH, ROW), jnp.int32),
                  pltpu.VMEM((ROW,), jnp.float32),
                  pltpu.SemaphoreType.DMA,
                  pltpu.SemaphoreType.DMA,
                  pltpu.SemaphoreType.DMA,
                  pltpu.SemaphoreType.DMA)


@functools.partial(
    pl.kernel,
    out_type=jax.ShapeDtypeStruct((NC, NP, 8), jnp.float32),
    mesh=_mesh,
    compiler_params=_sc_params,
    scratch_types=[
        pltpu.VMEM_SHARED((NP, 8), jnp.float32),
        pltpu.VMEM_SHARED((NP, 8), jnp.float32),
    ],
)
def _agg_kernel(src_hbm, dst_hbm, x_hbm, deg_hbm, agg_out, y_sh, s_sh):
    cid = lax.axis_index("c")
    sid = lax.axis_index("s")
    wid = cid * NS + sid
    row0 = wid * RPW
    sl1 = pl.ds(sid * NSL, NSL)
    sl2 = pl.ds(sid * NSL, NSL)
    iota = lax.iota(jnp.int32, 16)
    rbase = lax.shift_right_logical(iota, 3)
    cols = jnp.bitwise_and(iota, 7)

    def scoped(xb, dinvb, d0b, d1b, src_a, dst_a, src_b, dst_b, rows_a, rows_b,
               isem_a, isem_b, gsem_a, gsem_b, ssem_a, ssem_b):
        def dinv_pass(h):
            slh1 = pl.ds(sid * NSL + h * PR, PR)
            pltpu.sync_copy(deg_hbm.at[0, slh1], d0b)
            pltpu.sync_copy(deg_hbm.at[1, slh1], d1b)

            def dloop(v, carry):
                ds = pl.ds(v * 16, 16)
                dinvb[ds] = _newton_rsqrt(d0b[ds] + d1b[ds] + 1.0)
                return carry

            lax.fori_loop(0, PR // 16, dloop, 0)

        def stage1(h, carry):
            slh = pl.ds(sid * NSL + h * PR, PR)
            dinv_pass(h)
            pltpu.sync_copy(x_hbm.at[slh], xb)

            def yloop(j, carry):
                for k in range(4):
                    v = j * 4 + k
                    rows = rbase + 2 * v
                    xv = plsc.load_gather(xb, [rows, cols])
                    dv = plsc.load_gather(dinvb, [rows])
                    plsc.store_scatter(xb, [rows, cols], dv * xv)
                return carry

            lax.fori_loop(0, PV // 4, yloop, 0)
            pltpu.sync_copy(xb, y_sh.at[slh])

            @pl.when(cid == 1)
            def _():
                zv = jnp.zeros((16,), jnp.float32)

                def zloop(j, carry):
                    for k in range(4):
                        v = j * 4 + k
                        plsc.store_scatter(xb, [rbase + 2 * v, cols], zv)
                    return carry

                lax.fori_loop(0, PV // 4, zloop, 0)

            pltpu.sync_copy(xb, s_sh.at[slh])
            return carry

        lax.fori_loop(0, NPASS, stage1, 0)
        plsc.subcore_barrier()

        pltpu.sync_copy(src_hbm.at[pl.ds(row0, CH)], src_a)
        pltpu.sync_copy(dst_hbm.at[pl.ds(row0, CH)], dst_a)

        def body(j, carry):
            r = row0 + j * 2 * CH
            ib1 = pltpu.async_copy(src_hbm.at[pl.ds(r + CH, CH)], src_b, isem_b)
            ib2 = pltpu.async_copy(dst_hbm.at[pl.ds(r + CH, CH)], dst_b, isem_b)
            ga = [
                pltpu.async_copy(y_sh.at[src_a.at[i]], rows_a.at[i], gsem_a)
                for i in range(CH)
            ]
            for d in ga:
                d.wait()
            sa = [
                pltpu.async_copy(rows_a.at[i], s_sh.at[dst_a.at[i]], ssem_a,
                                 add=True)
                for i in range(CH)
            ]
            ib1.wait()
            ib2.wait()
            gb = [
                pltpu.async_copy(y_sh.at[src_b.at[i]], rows_b.at[i], gsem_b)
                for i in range(CH)
            ]
            for d in gb:
                d.wait()
            sb = [
                pltpu.async_copy(rows_b.at[i], s_sh.at[dst_b.at[i]], ssem_b,
                                 add=True)
                for i in range(CH)
            ]
            for d in sa:
                d.wait()
            nxt = row0 + ((2 * j + 2) % ITERS) * CH
            ia1 = pltpu.async_copy(src_hbm.at[pl.ds(nxt, CH)], src_a, isem_a)
            ia2 = pltpu.async_copy(dst_hbm.at[pl.ds(nxt, CH)], dst_a, isem_a)
            for d in sb:
                d.wait()
            ia1.wait()
            ia2.wait()
            return carry

        lax.fori_loop(0, ITERS // 2, body, 0)
        plsc.subcore_barrier()

        def stage3(h, carry):
            slh = pl.ds(sid * NSL + h * PR, PR)
            dinv_pass(h)
            pltpu.sync_copy(s_sh.at[slh], xb)

            def aloop(j, carry):
                for k in range(4):
                    v = j * 4 + k
                    rows = rbase + 2 * v
                    sv = plsc.load_gather(xb, [rows, cols])
                    dv = plsc.load_gather(dinvb, [rows])
                    plsc.store_scatter(xb, [rows, cols], dv * sv)
                return carry

            lax.fori_loop(0, PV // 4, aloop, 0)
            pltpu.sync_copy(xb, agg_out.at[cid, slh])
            return carry

        lax.fori_loop(0, NPASS, stage3, 0)

    pl.run_scoped(scoped,
                  pltpu.VMEM((PR, 8), jnp.float32),
                  pltpu.VMEM((PR,), jnp.float32),
                  pltpu.VMEM((PR,), jnp.float32),
                  pltpu.VMEM((PR,), jnp.float32),
                  pltpu.VMEM((CH, ROW), jnp.int32),
                  pltpu.VMEM((CH, ROW), jnp.int32),
                  pltpu.VMEM((CH, ROW), jnp.int32),
                  pltpu.VMEM((CH, ROW), jnp.int32),
                  pltpu.VMEM((CH, ROW, 8), jnp.float32),
                  pltpu.VMEM((CH, ROW, 8), jnp.float32),
                  pltpu.SemaphoreType.DMA,
                  pltpu.SemaphoreType.DMA,
                  pltpu.SemaphoreType.DMA,
                  pltpu.SemaphoreType.DMA,
                  pltpu.SemaphoreType.DMA,
                  pltpu.SemaphoreType.DMA)


def _hidden_body(a_ref, w_ref, b_ref, hid_ref):
    agg = a_ref[0] + a_ref[1]
    hid_ref[...] = (
        jnp.dot(agg, w_ref[...], preferred_element_type=jnp.float32) + b_ref[...]
    )


def _hidden(agg_parts, W, b2):
    blk = 2000
    grid = (N // blk,)
    return pl.pallas_call(
        _hidden_body,
        grid=grid,
        in_specs=[
            pl.BlockSpec((NC, blk, 8), lambda i: (0, i, 0)),
            pl.BlockSpec((8, 16), lambda i: (0, 0)),
            pl.BlockSpec((1, 16), lambda i: (0, 0)),
        ],
        out_specs=pl.BlockSpec((blk, 16), lambda i: (i, 0)),
        out_shape=jax.ShapeDtypeStruct((N, 16), jnp.float32),
    )(agg_parts, W, b2)


def _lin_body(h_ref, wl_ref, bl_ref, out_ref):
    h = jnp.maximum(h_ref[...], 0.0)
    out_ref[...] = (
        jnp.dot(h, wl_ref[...], preferred_element_type=jnp.float32) + bl_ref[...]
    )


def _linear(h2, Wl, bl2):
    blk = 2000
    grid = (N // 5 // blk,)
    return pl.pallas_call(
        _lin_body,
        grid=grid,
        in_specs=[
            pl.BlockSpec((blk, 80), lambda i: (i, 0)),
            pl.BlockSpec((80, 445), lambda i: (0, 0)),
            pl.BlockSpec((1, 445), lambda i: (0, 0)),
        ],
        out_specs=pl.BlockSpec((blk, 445), lambda i: (i, 0)),
        out_shape=jax.ShapeDtypeStruct((N // 5, 445), jnp.float32),
    )(h2, Wl, bl2)


def kernel(x, edge_index, W, b, Wl, bl):
    src = edge_index[0]
    dst = edge_index[1]
    pad = jnp.full((E_PAD - E,), TRASH, dtype=jnp.int32)
    src_p = jnp.concatenate([src, pad]).reshape(ROWS_TOT, ROW)
    dst_p = jnp.concatenate([dst, pad]).reshape(ROWS_TOT, ROW)
    x_pad = jnp.pad(x, ((0, NP - N), (0, 0)))

    deg_parts = _deg_kernel(dst_p)
    agg_parts = _agg_kernel(src_p, dst_p, x_pad, deg_parts)
    hidden = _hidden(agg_parts, W, b.reshape(1, 16))
    h2 = hidden.reshape(N // 5, 80)
    out = _linear(h2, Wl, bl.reshape(1, 445)).reshape(N // 5, 89, 5)
    return (out, hidden)


# speedup vs baseline: 151.5060x; 1.0176x over previous
import functools

import jax
import jax.numpy as jnp
from jax import lax
from jax.experimental import pallas as pl
from jax.experimental.pallas import tpu as pltpu
from jax.experimental.pallas import tpu_sc as plsc

N = 100000
NP = 102400
TRASH = N
E = 6400000
NC, NS = 2, 16
NW = NC * NS
ROW = 128
RPW = 1568
CH = 7
ITERS = RPW // CH
ROWS_TOT = NW * RPW
E_PAD = ROWS_TOT * ROW
NSL = NP // NS
NV = NSL // 16
NV8 = NSL * 8 // 16
NPASS = 16
PR = NSL // NPASS
PV = PR * 8 // 16

_mesh = plsc.VectorSubcoreMesh(core_axis_name="c", subcore_axis_name="s")
_sc_params = pltpu.CompilerParams(use_tc_tiling_on_sc=False,
                                  needs_layout_passes=False)


def _newton_rsqrt(d):
    u = plsc.bitcast(d, jnp.int32)
    u = 0x5F3759DF - lax.shift_right_logical(u, 1)
    g = plsc.bitcast(u, jnp.float32)
    for _ in range(3):
        g = g * (1.5 - 0.5 * d * g * g)
    return g


@functools.partial(
    pl.kernel,
    out_type=jax.ShapeDtypeStruct((NC, NP), jnp.float32),
    mesh=_mesh,
    compiler_params=_sc_params,
    scratch_types=[
        pltpu.VMEM_SHARED((NP,), jnp.float32),
    ],
)
def _deg_kernel(dst_hbm, deg_out, deg_sh):
    cid = lax.axis_index("c")
    sid = lax.axis_index("s")
    wid = cid * NS + sid
    row0 = wid * RPW

    def scoped(zb, idx_a, idx_b, ones_v, isem_a, isem_b, sem_a, sem_b):
        zv = jnp.zeros((16,), jnp.float32)

        def zinit(v, carry):
            zb[pl.ds(v * 16, 16)] = zv
            return carry

        lax.fori_loop(0, NV, zinit, 0)
        for i in range(8):
            ones_v[pl.ds(i * 16, 16)] = zv + 1.0
        pltpu.sync_copy(zb, deg_sh.at[pl.ds(sid * NSL, NSL)])
        plsc.subcore_barrier()

        pltpu.sync_copy(dst_hbm.at[pl.ds(row0, CH)], idx_a)

        def body(j, carry):
            r = row0 + j * 2 * CH
            ib = pltpu.async_copy(dst_hbm.at[pl.ds(r + CH, CH)], idx_b, isem_b)
            sa = [
                pltpu.async_copy(ones_v, deg_sh.at[idx_a.at[i]], sem_a, add=True)
                for i in range(CH)
            ]
            ib.wait()
            sb = [
                pltpu.async_copy(ones_v, deg_sh.at[idx_b.at[i]], sem_b, add=True)
                for i in range(CH)
            ]
            for d in sa:
                d.wait()
            nxt = row0 + ((2 * j + 2) % ITERS) * CH
            ia = pltpu.async_copy(dst_hbm.at[pl.ds(nxt, CH)], idx_a, isem_a)
            for d in sb:
                d.wait()
            ia.wait()
            return carry

        lax.fori_loop(0, ITERS // 2, body, 0)
        plsc.subcore_barrier()
        pltpu.sync_copy(deg_sh.at[pl.ds(sid * NSL, NSL)],
                        deg_out.at[cid, pl.ds(sid * NSL, NSL)])

    pl.run_scoped(scoped,
                  pltpu.VMEM((NSL,), jnp.float32),
                  pltpu.VMEM((CH, ROW), jnp.int32),
                  pltpu.VMEM((CH, ROW), jnp.int32),
                  pltpu.VMEM((ROW,), jnp.float32),
                  pltpu.SemaphoreType.DMA,
                  pltpu.SemaphoreType.DMA,
                  pltpu.SemaphoreType.DMA,
                  pltpu.SemaphoreType.DMA)


@functools.partial(
    pl.kernel,
    out_type=jax.ShapeDtypeStruct((NC, NP, 8), jnp.float32),
    mesh=_mesh,
    compiler_params=_sc_params,
    scratch_types=[
        pltpu.VMEM_SHARED((NP, 8), jnp.float32),
        pltpu.VMEM_SHARED((NP, 8), jnp.float32),
    ],
)
def _agg_kernel(src_hbm, dst_hbm, x_hbm, deg_hbm, agg_out, y_sh, s_sh):
    cid = lax.axis_index("c")
    sid = lax.axis_index("s")
    wid = cid * NS + sid
    row0 = wid * RPW
    sl1 = pl.ds(sid * NSL, NSL)
    sl2 = pl.ds(sid * NSL, NSL)
    iota = lax.iota(jnp.int32, 16)
    rbase = lax.shift_right_logical(iota, 3)
    cols = jnp.bitwise_and(iota, 7)

    def scoped(xb, dinvb, d0b, d1b, src_a, dst_a, src_b, dst_b, rows_a, rows_b,
               isem_a, isem_b, gsem_a, gsem_b, ssem_a, ssem_b):
        def dinv_pass(h):
            slh1 = pl.ds(sid * NSL + h * PR, PR)
            pltpu.sync_copy(deg_hbm.at[0, slh1], d0b)
            pltpu.sync_copy(deg_hbm.at[1, slh1], d1b)

            def dloop(v, carry):
                ds = pl.ds(v * 16, 16)
                dinvb[ds] = _newton_rsqrt(d0b[ds] + d1b[ds] + 1.0)
                return carry

            lax.fori_loop(0, PR // 16, dloop, 0)

        def stage1(h, carry):
            slh = pl.ds(sid * NSL + h * PR, PR)
            dinv_pass(h)
            pltpu.sync_copy(x_hbm.at[slh], xb)

            def yloop(j, carry):
                for k in range(4):
                    v = j * 4 + k
                    rows = rbase + 2 * v
                    xv = plsc.load_gather(xb, [rows, cols])
                    dv = plsc.load_gather(dinvb, [rows])
                    plsc.store_scatter(xb, [rows, cols], dv * xv)
                return carry

            lax.fori_loop(0, PV // 4, yloop, 0)
            pltpu.sync_copy(xb, y_sh.at[slh])

            @pl.when(cid == 1)
            def _():
                zv = jnp.zeros((16,), jnp.float32)

                def zloop(j, carry):
                    for k in range(4):
                        v = j * 4 + k
                        plsc.store_scatter(xb, [rbase + 2 * v, cols], zv)
                    return carry

                lax.fori_loop(0, PV // 4, zloop, 0)

            pltpu.sync_copy(xb, s_sh.at[slh])
            return carry

        lax.fori_loop(0, NPASS, stage1, 0)
        plsc.subcore_barrier()

        def fire_g(srcb, rowsb, sem):
            return [
                pltpu.async_copy(y_sh.at[srcb.at[i]], rowsb.at[i], sem)
                for i in range(CH)
            ]

        def fire_s(rowsb, dstb, sem):
            return [
                pltpu.async_copy(rowsb.at[i], s_sh.at[dstb.at[i]], sem,
                                 add=True)
                for i in range(CH)
            ]

        pltpu.sync_copy(src_hbm.at[pl.ds(row0, CH)], src_b)
        pltpu.sync_copy(dst_hbm.at[pl.ds(row0, CH)], dst_b)
        for d in fire_g(src_b, rows_b, gsem_b):
            d.wait()
        fire_s(rows_b, dst_b, ssem_b)
        pltpu.sync_copy(src_hbm.at[pl.ds(row0 + CH, CH)], src_a)
        pltpu.sync_copy(dst_hbm.at[pl.ds(row0 + CH, CH)], dst_a)

        def body(j, carry):
            r = row0 + (2 * j + 1) * CH
            ga = fire_g(src_a, rows_a, gsem_a)
            for i in range(CH):
                pltpu.make_async_copy(rows_b.at[i], s_sh.at[dst_b.at[i]],
                                      ssem_b).wait()
            ib1 = pltpu.async_copy(src_hbm.at[pl.ds(r + CH, CH)], src_b, isem_b)
            ib2 = pltpu.async_copy(dst_hbm.at[pl.ds(r + CH, CH)], dst_b, isem_b)
            for d in ga:
                d.wait()
            sa = fire_s(rows_a, dst_a, ssem_a)
            ib1.wait()
            ib2.wait()
            gb = fire_g(src_b, rows_b, gsem_b)
            for d in sa:
                d.wait()
            nxt = row0 + ((2 * j + 3) % ITERS) * CH
            ia1 = pltpu.async_copy(src_hbm.at[pl.ds(nxt, CH)], src_a, isem_a)
            ia2 = pltpu.async_copy(dst_hbm.at[pl.ds(nxt, CH)], dst_a, isem_a)
            for d in gb:
                d.wait()
            fire_s(rows_b, dst_b, ssem_b)
            ia1.wait()
            ia2.wait()
            return carry

        lax.fori_loop(0, (ITERS - 2) // 2, body, 0)

        for i in range(CH):
            pltpu.make_async_copy(rows_b.at[i], s_sh.at[dst_b.at[i]],
                                  ssem_b).wait()
        for d in fire_g(src_a, rows_a, gsem_a):
            d.wait()
        for d in fire_s(rows_a, dst_a, ssem_a):
            d.wait()
        plsc.subcore_barrier()

        def stage3(h, carry):
            slh = pl.ds(sid * NSL + h * PR, PR)
            dinv_pass(h)
            pltpu.sync_copy(s_sh.at[slh], xb)

            def aloop(j, carry):
                for k in range(4):
                    v = j * 4 + k
                    rows = rbase + 2 * v
                    sv = plsc.load_gather(xb, [rows, cols])
                    dv = plsc.load_gather(dinvb, [rows])
                    plsc.store_scatter(xb, [rows, cols], dv * sv)
                return carry

            lax.fori_loop(0, PV // 4, aloop, 0)
            pltpu.sync_copy(xb, agg_out.at[cid, slh])
            return carry

        lax.fori_loop(0, NPASS, stage3, 0)

    pl.run_scoped(scoped,
                  pltpu.VMEM((PR, 8), jnp.float32),
                  pltpu.VMEM((PR,), jnp.float32),
                  pltpu.VMEM((PR,), jnp.float32),
                  pltpu.VMEM((PR,), jnp.float32),
                  pltpu.VMEM((CH, ROW), jnp.int32),
                  pltpu.VMEM((CH, ROW), jnp.int32),
                  pltpu.VMEM((CH, ROW), jnp.int32),
                  pltpu.VMEM((CH, ROW), jnp.int32),
                  pltpu.VMEM((CH, ROW, 8), jnp.float32),
                  pltpu.VMEM((CH, ROW, 8), jnp.float32),
                  pltpu.SemaphoreType.DMA,
                  pltpu.SemaphoreType.DMA,
                  pltpu.SemaphoreType.DMA,
                  pltpu.SemaphoreType.DMA,
                  pltpu.SemaphoreType.DMA,
                  pltpu.SemaphoreType.DMA)


def _hidden_body(a_ref, w_ref, b_ref, hid_ref):
    agg = a_ref[0] + a_ref[1]
    hid_ref[...] = (
        jnp.dot(agg, w_ref[...], preferred_element_type=jnp.float32) + b_ref[...]
    )


def _hidden(agg_parts, W, b2):
    blk = 2000
    grid = (N // blk,)
    return pl.pallas_call(
        _hidden_body,
        grid=grid,
        in_specs=[
            pl.BlockSpec((NC, blk, 8), lambda i: (0, i, 0)),
            pl.BlockSpec((8, 16), lambda i: (0, 0)),
            pl.BlockSpec((1, 16), lambda i: (0, 0)),
        ],
        out_specs=pl.BlockSpec((blk, 16), lambda i: (i, 0)),
        out_shape=jax.ShapeDtypeStruct((N, 16), jnp.float32),
    )(agg_parts, W, b2)


def _lin_body(h_ref, wl_ref, bl_ref, out_ref):
    h = jnp.maximum(h_ref[...], 0.0)
    out_ref[...] = (
        jnp.dot(h, wl_ref[...], preferred_element_type=jnp.float32) + bl_ref[...]
    )


def _linear(h2, Wl, bl2):
    blk = 2000
    grid = (N // 5 // blk,)
    return pl.pallas_call(
        _lin_body,
        grid=grid,
        in_specs=[
            pl.BlockSpec((blk, 80), lambda i: (i, 0)),
            pl.BlockSpec((80, 445), lambda i: (0, 0)),
            pl.BlockSpec((1, 445), lambda i: (0, 0)),
        ],
        out_specs=pl.BlockSpec((blk, 445), lambda i: (i, 0)),
        out_shape=jax.ShapeDtypeStruct((N // 5, 445), jnp.float32),
    )(h2, Wl, bl2)


def kernel(x, edge_index, W, b, Wl, bl):
    src = edge_index[0]
    dst = edge_index[1]
    pad = jnp.full((E_PAD - E,), TRASH, dtype=jnp.int32)
    src_p = jnp.concatenate([src, pad]).reshape(ROWS_TOT, ROW)
    dst_p = jnp.concatenate([dst, pad]).reshape(ROWS_TOT, ROW)
    x_pad = jnp.pad(x, ((0, NP - N), (0, 0)))

    deg_parts = _deg_kernel(dst_p)
    agg_parts = _agg_kernel(src_p, dst_p, x_pad, deg_parts)
    hidden = _hidden(agg_parts, W, b.reshape(1, 16))
    h2 = hidden.reshape(N // 5, 80)
    out = _linear(h2, Wl, bl.reshape(1, 445)).reshape(N // 5, 89, 5)
    return (out, hidden)


# speedup vs baseline: 152.0152x; 1.0034x over previous
import functools

import jax
import jax.numpy as jnp
from jax import lax
from jax.experimental import pallas as pl
from jax.experimental.pallas import tpu as pltpu
from jax.experimental.pallas import tpu_sc as plsc

N = 100000
NP = 102400
TRASH = N
E = 6400000
NC, NS = 2, 16
NW = NC * NS
ROW = 128
RPW = 1568
CH = 7
ITERS = RPW // CH
ROWS_TOT = NW * RPW
E_PAD = ROWS_TOT * ROW
NSL = NP // NS
NV = NSL // 16
NV8 = NSL * 8 // 16
NPASS = 16
PR = NSL // NPASS
PV = PR * 8 // 16

_mesh = plsc.VectorSubcoreMesh(core_axis_name="c", subcore_axis_name="s")
_sc_params = pltpu.CompilerParams(use_tc_tiling_on_sc=False,
                                  needs_layout_passes=False)


def _newton_rsqrt(d):
    u = plsc.bitcast(d, jnp.int32)
    u = 0x5F3759DF - lax.shift_right_logical(u, 1)
    g = plsc.bitcast(u, jnp.float32)
    for _ in range(3):
        g = g * (1.5 - 0.5 * d * g * g)
    return g


@functools.partial(
    pl.kernel,
    out_type=jax.ShapeDtypeStruct((NC, NP), jnp.float32),
    mesh=_mesh,
    compiler_params=_sc_params,
    scratch_types=[
        pltpu.VMEM_SHARED((NP,), jnp.float32),
    ],
)
def _deg_kernel(dst_hbm, deg_out, deg_sh):
    cid = lax.axis_index("c")
    sid = lax.axis_index("s")
    wid = cid * NS + sid
    row0 = wid * RPW

    e0 = row0 * ROW
    CE = CH * ROW

    def scoped(zb, idx_a, idx_b, ones_v, isem_a, isem_b, sem_a, sem_b):
        zv = jnp.zeros((16,), jnp.float32)

        def zinit(v, carry):
            zb[pl.ds(v * 16, 16)] = zv
            return carry

        lax.fori_loop(0, NV, zinit, 0)

        def oinit(v, carry):
            ones_v[pl.ds(v * 16, 16)] = zv + 1.0
            return carry

        lax.fori_loop(0, CE // 16, oinit, 0)
        pltpu.sync_copy(zb, deg_sh.at[pl.ds(sid * NSL, NSL)])
        plsc.subcore_barrier()

        pltpu.sync_copy(dst_hbm.at[pl.ds(e0, CE)], idx_a)

        def body(j, carry):
            r = e0 + j * 2 * CE
            ib = pltpu.async_copy(dst_hbm.at[pl.ds(r + CE, CE)], idx_b, isem_b)
            sa = pltpu.async_copy(ones_v, deg_sh.at[idx_a], sem_a, add=True)
            ib.wait()
            sb = pltpu.async_copy(ones_v, deg_sh.at[idx_b], sem_b, add=True)
            sa.wait()
            nxt = e0 + ((2 * j + 2) % ITERS) * CE
            ia = pltpu.async_copy(dst_hbm.at[pl.ds(nxt, CE)], idx_a, isem_a)
            sb.wait()
            ia.wait()
            return carry

        lax.fori_loop(0, ITERS // 2, body, 0)
        plsc.subcore_barrier()
        pltpu.sync_copy(deg_sh.at[pl.ds(sid * NSL, NSL)],
                        deg_out.at[cid, pl.ds(sid * NSL, NSL)])

    pl.run_scoped(scoped,
                  pltpu.VMEM((NSL,), jnp.float32),
                  pltpu.VMEM((CH * ROW,), jnp.int32),
                  pltpu.VMEM((CH * ROW,), jnp.int32),
                  pltpu.VMEM((CH * ROW,), jnp.float32),
                  pltpu.SemaphoreType.DMA,
                  pltpu.SemaphoreType.DMA,
                  pltpu.SemaphoreType.DMA,
                  pltpu.SemaphoreType.DMA)


@functools.partial(
    pl.kernel,
    out_type=jax.ShapeDtypeStruct((NC, NP, 8), jnp.float32),
    mesh=_mesh,
    compiler_params=_sc_params,
    scratch_types=[
        pltpu.VMEM_SHARED((NP, 8), jnp.float32),
        pltpu.VMEM_SHARED((NP, 8), jnp.float32),
    ],
)
def _agg_kernel(src_hbm, dst_hbm, x_hbm, deg_hbm, agg_out, y_sh, s_sh):
    cid = lax.axis_index("c")
    sid = lax.axis_index("s")
    wid = cid * NS + sid
    row0 = wid * RPW
    sl1 = pl.ds(sid * NSL, NSL)
    sl2 = pl.ds(sid * NSL, NSL)
    iota = lax.iota(jnp.int32, 16)
    rbase = lax.shift_right_logical(iota, 3)
    cols = jnp.bitwise_and(iota, 7)

    def scoped(xb, dinvb, d0b, d1b, src_a, dst_a, src_b, dst_b, rows_a, rows_b,
               isem_a, isem_b, gsem_a, gsem_b, ssem_a, ssem_b):
        def dinv_pass(h):
            slh1 = pl.ds(sid * NSL + h * PR, PR)
            pltpu.sync_copy(deg_hbm.at[0, slh1], d0b)
            pltpu.sync_copy(deg_hbm.at[1, slh1], d1b)

            def dloop(v, carry):
                ds = pl.ds(v * 16, 16)
                dinvb[ds] = _newton_rsqrt(d0b[ds] + d1b[ds] + 1.0)
                return carry

            lax.fori_loop(0, PR // 16, dloop, 0)

        def stage1(h, carry):
            slh = pl.ds(sid * NSL + h * PR, PR)
            dinv_pass(h)
            pltpu.sync_copy(x_hbm.at[slh], xb)

            def yloop(j, carry):
                for k in range(4):
                    v = j * 4 + k
                    rows = rbase + 2 * v
                    xv = plsc.load_gather(xb, [rows, cols])
                    dv = plsc.load_gather(dinvb, [rows])
                    plsc.store_scatter(xb, [rows, cols], dv * xv)
                return carry

            lax.fori_loop(0, PV // 4, yloop, 0)
            pltpu.sync_copy(xb, y_sh.at[slh])

            @pl.when(cid == 1)
            def _():
                zv = jnp.zeros((16,), jnp.float32)

                def zloop(j, carry):
                    for k in range(4):
                        v = j * 4 + k
                        plsc.store_scatter(xb, [rbase + 2 * v, cols], zv)
                    return carry

                lax.fori_loop(0, PV // 4, zloop, 0)

            pltpu.sync_copy(xb, s_sh.at[slh])
            return carry

        lax.fori_loop(0, NPASS, stage1, 0)
        plsc.subcore_barrier()

        e0 = row0 * ROW
        CE = CH * ROW

        pltpu.sync_copy(src_hbm.at[pl.ds(e0, CE)], src_b)
        pltpu.sync_copy(dst_hbm.at[pl.ds(e0, CE)], dst_b)
        pltpu.async_copy(y_sh.at[src_b], rows_b, gsem_b).wait()
        pltpu.async_copy(rows_b, s_sh.at[dst_b], ssem_b, add=True)
        pltpu.sync_copy(src_hbm.at[pl.ds(e0 + CE, CE)], src_a)
        pltpu.sync_copy(dst_hbm.at[pl.ds(e0 + CE, CE)], dst_a)

        def body(j, carry):
            r = e0 + (2 * j + 1) * CE
            ga = pltpu.async_copy(y_sh.at[src_a], rows_a, gsem_a)
            pltpu.make_async_copy(rows_b, s_sh.at[dst_b], ssem_b).wait()
            ib1 = pltpu.async_copy(src_hbm.at[pl.ds(r + CE, CE)], src_b, isem_b)
            ib2 = pltpu.async_copy(dst_hbm.at[pl.ds(r + CE, CE)], dst_b, isem_b)
            ga.wait()
            sa = pltpu.async_copy(rows_a, s_sh.at[dst_a], ssem_a, add=True)
            ib1.wait()
            ib2.wait()
            gb = pltpu.async_copy(y_sh.at[src_b], rows_b, gsem_b)
            sa.wait()
            nxt = e0 + ((2 * j + 3) % ITERS) * CE
            ia1 = pltpu.async_copy(src_hbm.at[pl.ds(nxt, CE)], src_a, isem_a)
            ia2 = pltpu.async_copy(dst_hbm.at[pl.ds(nxt, CE)], dst_a, isem_a)
            gb.wait()
            pltpu.async_copy(rows_b, s_sh.at[dst_b], ssem_b, add=True)
            ia1.wait()
            ia2.wait()
            return carry

        lax.fori_loop(0, (ITERS - 2) // 2, body, 0)

        pltpu.make_async_copy(rows_b, s_sh.at[dst_b], ssem_b).wait()
        pltpu.async_copy(y_sh.at[src_a], rows_a, gsem_a).wait()
        pltpu.async_copy(rows_a, s_sh.at[dst_a], ssem_a, add=True).wait()
        plsc.subcore_barrier()

        def stage3(h, carry):
            slh = pl.ds(sid * NSL + h * PR, PR)
            dinv_pass(h)
            pltpu.sync_copy(s_sh.at[slh], xb)

            def aloop(j, carry):
                for k in range(4):
                    v = j * 4 + k
                    rows = rbase + 2 * v
                    sv = plsc.load_gather(xb, [rows, cols])
                    dv = plsc.load_gather(dinvb, [rows])
                    plsc.store_scatter(xb, [rows, cols], dv * sv)
                return carry

            lax.fori_loop(0, PV // 4, aloop, 0)
            pltpu.sync_copy(xb, agg_out.at[cid, slh])
            return carry

        lax.fori_loop(0, NPASS, stage3, 0)

    pl.run_scoped(scoped,
                  pltpu.VMEM((PR, 8), jnp.float32),
                  pltpu.VMEM((PR,), jnp.float32),
                  pltpu.VMEM((PR,), jnp.float32),
                  pltpu.VMEM((PR,), jnp.float32),
                  pltpu.VMEM((CH * ROW,), jnp.int32),
                  pltpu.VMEM((CH * ROW,), jnp.int32),
                  pltpu.VMEM((CH * ROW,), jnp.int32),
                  pltpu.VMEM((CH * ROW,), jnp.int32),
                  pltpu.VMEM((CH * ROW, 8), jnp.float32),
                  pltpu.VMEM((CH * ROW, 8), jnp.float32),
                  pltpu.SemaphoreType.DMA,
                  pltpu.SemaphoreType.DMA,
                  pltpu.SemaphoreType.DMA,
                  pltpu.SemaphoreType.DMA,
                  pltpu.SemaphoreType.DMA,
                  pltpu.SemaphoreType.DMA)


def _hidden_body(a_ref, w_ref, b_ref, hid_ref):
    agg = a_ref[0] + a_ref[1]
    hid_ref[...] = (
        jnp.dot(agg, w_ref[...], preferred_element_type=jnp.float32) + b_ref[...]
    )


def _hidden(agg_parts, W, b2):
    blk = 2000
    grid = (N // blk,)
    return pl.pallas_call(
        _hidden_body,
        grid=grid,
        in_specs=[
            pl.BlockSpec((NC, blk, 8), lambda i: (0, i, 0)),
            pl.BlockSpec((8, 16), lambda i: (0, 0)),
            pl.BlockSpec((1, 16), lambda i: (0, 0)),
        ],
        out_specs=pl.BlockSpec((blk, 16), lambda i: (i, 0)),
        out_shape=jax.ShapeDtypeStruct((N, 16), jnp.float32),
    )(agg_parts, W, b2)


def _lin_body(h_ref, wl_ref, bl_ref, out_ref):
    h = jnp.maximum(h_ref[...], 0.0)
    out_ref[...] = (
        jnp.dot(h, wl_ref[...], preferred_element_type=jnp.float32) + bl_ref[...]
    )


def _linear(h2, Wl, bl2):
    blk = 2000
    grid = (N // 5 // blk,)
    return pl.pallas_call(
        _lin_body,
        grid=grid,
        in_specs=[
            pl.BlockSpec((blk, 80), lambda i: (i, 0)),
            pl.BlockSpec((80, 445), lambda i: (0, 0)),
            pl.BlockSpec((1, 445), lambda i: (0, 0)),
        ],
        out_specs=pl.BlockSpec((blk, 445), lambda i: (i, 0)),
        out_shape=jax.ShapeDtypeStruct((N // 5, 445), jnp.float32),
    )(h2, Wl, bl2)


def kernel(x, edge_index, W, b, Wl, bl):
    src = edge_index[0]
    dst = edge_index[1]
    pad = jnp.full((E_PAD - E,), TRASH, dtype=jnp.int32)
    src_p = jnp.concatenate([src, pad])
    dst_p = jnp.concatenate([dst, pad])
    x_pad = jnp.pad(x, ((0, NP - N), (0, 0)))

    deg_parts = _deg_kernel(dst_p)
    agg_parts = _agg_kernel(src_p, dst_p, x_pad, deg_parts)
    hidden = _hidden(agg_parts, W, b.reshape(1, 16))
    h2 = hidden.reshape(N // 5, 80)
    out = _linear(h2, Wl, bl.reshape(1, 445)).reshape(N // 5, 89, 5)
    return (out, hidden)


# speedup vs baseline: 152.0502x; 1.0002x over previous
import functools

import jax
import jax.numpy as jnp
from jax import lax
from jax.experimental import pallas as pl
from jax.experimental.pallas import tpu as pltpu
from jax.experimental.pallas import tpu_sc as plsc

N = 100000
NP = 102400
TRASH = N
E = 6400000
NC, NS = 2, 16
NW = NC * NS
ROW = 128
RPW = 1568
CH = 7
ITERS = RPW // CH
ROWS_TOT = NW * RPW
E_PAD = ROWS_TOT * ROW
NSL = NP // NS
NV = NSL // 16
NPASS = 16
PR = NSL // NPASS
PV = PR * 8 // 16

_mesh = plsc.VectorSubcoreMesh(core_axis_name="c", subcore_axis_name="s")
_sc_params = pltpu.CompilerParams(use_tc_tiling_on_sc=False,
                                  needs_layout_passes=False)


def _newton_rsqrt(d):
    u = plsc.bitcast(d, jnp.int32)
    u = 0x5F3759DF - lax.shift_right_logical(u, 1)
    g = plsc.bitcast(u, jnp.float32)
    for _ in range(3):
        g = g * (1.5 - 0.5 * d * g * g)
    return g


@functools.partial(
    pl.kernel,
    out_type=jax.ShapeDtypeStruct((NC, NP), jnp.float32),
    mesh=_mesh,
    compiler_params=_sc_params,
    scratch_types=[
        pltpu.VMEM_SHARED((NP,), jnp.float32),
    ],
)
def _deg_kernel(dst_hbm, deg_out, deg_sh):
    cid = lax.axis_index("c")
    sid = lax.axis_index("s")
    wid = cid * NS + sid
    row0 = wid * RPW

    e0 = row0 * ROW
    CE = CH * ROW

    def scoped(zb, idx_a, idx_b, ones_v, isem_a, isem_b, sem_a, sem_b):
        zv = jnp.zeros((16,), jnp.float32)

        def zinit(v, carry):
            zb[pl.ds(v * 16, 16)] = zv
            return carry

        lax.fori_loop(0, NV, zinit, 0)

        def oinit(v, carry):
            ones_v[pl.ds(v * 16, 16)] = zv + 1.0
            return carry

        lax.fori_loop(0, CE // 16, oinit, 0)
        pltpu.sync_copy(zb, deg_sh.at[pl.ds(sid * NSL, NSL)])
        plsc.subcore_barrier()

        pltpu.sync_copy(dst_hbm.at[pl.ds(e0, CE)], idx_a)

        def body(j, carry):
            r = e0 + j * 2 * CE
            ib = pltpu.async_copy(dst_hbm.at[pl.ds(r + CE, CE)], idx_b, isem_b)
            sa = pltpu.async_copy(ones_v, deg_sh.at[idx_a], sem_a, add=True)
            ib.wait()
            sb = pltpu.async_copy(ones_v, deg_sh.at[idx_b], sem_b, add=True)
            sa.wait()
            nxt = e0 + ((2 * j + 2) % ITERS) * CE
            ia = pltpu.async_copy(dst_hbm.at[pl.ds(nxt, CE)], idx_a, isem_a)
            sb.wait()
            ia.wait()
            return carry

        lax.fori_loop(0, ITERS // 2, body, 0)
        plsc.subcore_barrier()
        pltpu.sync_copy(deg_sh.at[pl.ds(sid * NSL, NSL)],
                        deg_out.at[cid, pl.ds(sid * NSL, NSL)])

    pl.run_scoped(scoped,
                  pltpu.VMEM((NSL,), jnp.float32),
                  pltpu.VMEM((CH * ROW,), jnp.int32),
                  pltpu.VMEM((CH * ROW,), jnp.int32),
                  pltpu.VMEM((CH * ROW,), jnp.float32),
                  pltpu.SemaphoreType.DMA,
                  pltpu.SemaphoreType.DMA,
                  pltpu.SemaphoreType.DMA,
                  pltpu.SemaphoreType.DMA)


@functools.partial(
    pl.kernel,
    out_type=jax.ShapeDtypeStruct((NC, NP, 8), jnp.float32),
    mesh=_mesh,
    compiler_params=_sc_params,
    scratch_types=[
        pltpu.VMEM_SHARED((NP, 8), jnp.float32),
        pltpu.VMEM_SHARED((NP, 8), jnp.float32),
    ],
)
def _agg_kernel(src_hbm, dst_hbm, x_hbm, deg_hbm, agg_out, y_sh, s_sh):
    cid = lax.axis_index("c")
    sid = lax.axis_index("s")
    wid = cid * NS + sid
    row0 = wid * RPW
    sl1 = pl.ds(sid * NSL, NSL)
    sl2 = pl.ds(sid * NSL, NSL)
    iota = lax.iota(jnp.int32, 16)
    rbase = lax.shift_right_logical(iota, 3)
    cols = jnp.bitwise_and(iota, 7)

    def scoped(xb, dinvb, d0b, d1b, src_a, dst_a, src_b, dst_b, rows_a, rows_b,
               isem_a, isem_b, gsem_a, gsem_b, ssem_a, ssem_b):
        def dinv_pass(h):
            slh1 = pl.ds(sid * NSL + h * PR, PR)
            pltpu.sync_copy(deg_hbm.at[0, slh1], d0b)
            pltpu.sync_copy(deg_hbm.at[1, slh1], d1b)

            def dloop(v, carry):
                ds = pl.ds(v * 16, 16)
                dinvb[ds] = _newton_rsqrt(d0b[ds] + d1b[ds] + 1.0)
                return carry

            lax.fori_loop(0, PR // 16, dloop, 0)

        def stage1(h, carry):
            slh = pl.ds(sid * NSL + h * PR, PR)
            dinv_pass(h)
            pltpu.sync_copy(x_hbm.at[slh], xb)

            def yloop(j, carry):
                for k in range(4):
                    v = j * 4 + k
                    rows = rbase + 2 * v
                    xv = plsc.load_gather(xb, [rows, cols])
                    dv = plsc.load_gather(dinvb, [rows])
                    plsc.store_scatter(xb, [rows, cols], dv * xv)
                return carry

            lax.fori_loop(0, PV // 4, yloop, 0)
            pltpu.sync_copy(xb, y_sh.at[slh])

            @pl.when(cid == 1)
            def _():
                zv = jnp.zeros((16,), jnp.float32)

                def zloop(j, carry):
                    for k in range(4):
                        v = j * 4 + k
                        plsc.store_scatter(xb, [rbase + 2 * v, cols], zv)
                    return carry

                lax.fori_loop(0, PV // 4, zloop, 0)

            pltpu.sync_copy(xb, s_sh.at[slh])
            return carry

        lax.fori_loop(0, NPASS, stage1, 0)
        plsc.subcore_barrier()

        e0 = row0 * ROW
        CE = CH * ROW

        pltpu.sync_copy(src_hbm.at[pl.ds(e0, CE)], src_b)
        pltpu.sync_copy(dst_hbm.at[pl.ds(e0, CE)], dst_b)
        pltpu.async_copy(y_sh.at[src_b], rows_b, gsem_b).wait()
        pltpu.async_copy(rows_b, s_sh.at[dst_b], ssem_b, add=True)
        pltpu.sync_copy(src_hbm.at[pl.ds(e0 + CE, CE)], src_a)
        pltpu.sync_copy(dst_hbm.at[pl.ds(e0 + CE, CE)], dst_a)

        def body(j, carry):
            r = e0 + (2 * j + 1) * CE
            ga = pltpu.async_copy(y_sh.at[src_a], rows_a, gsem_a)
            pltpu.make_async_copy(rows_b, s_sh.at[dst_b], ssem_b).wait()
            ib1 = pltpu.async_copy(src_hbm.at[pl.ds(r + CE, CE)], src_b, isem_b)
            ib2 = pltpu.async_copy(dst_hbm.at[pl.ds(r + CE, CE)], dst_b, isem_b)
            ga.wait()
            sa = pltpu.async_copy(rows_a, s_sh.at[dst_a], ssem_a, add=True)
            ib1.wait()
            ib2.wait()
            gb = pltpu.async_copy(y_sh.at[src_b], rows_b, gsem_b)
            sa.wait()
            nxt = e0 + ((2 * j + 3) % ITERS) * CE
            ia1 = pltpu.async_copy(src_hbm.at[pl.ds(nxt, CE)], src_a, isem_a)
            ia2 = pltpu.async_copy(dst_hbm.at[pl.ds(nxt, CE)], dst_a, isem_a)
            gb.wait()
            pltpu.async_copy(rows_b, s_sh.at[dst_b], ssem_b, add=True)
            ia1.wait()
            ia2.wait()
            return carry

        lax.fori_loop(0, (ITERS - 2) // 2, body, 0)

        pltpu.make_async_copy(rows_b, s_sh.at[dst_b], ssem_b).wait()
        pltpu.async_copy(y_sh.at[src_a], rows_a, gsem_a).wait()
        pltpu.async_copy(rows_a, s_sh.at[dst_a], ssem_a, add=True).wait()
        plsc.subcore_barrier()

        def stage3(h, carry):
            slh = pl.ds(sid * NSL + h * PR, PR)
            dinv_pass(h)
            pltpu.sync_copy(s_sh.at[slh], xb)

            def aloop(j, carry):
                for k in range(4):
                    v = j * 4 + k
                    rows = rbase + 2 * v
                    sv = plsc.load_gather(xb, [rows, cols])
                    dv = plsc.load_gather(dinvb, [rows])
                    plsc.store_scatter(xb, [rows, cols], dv * sv)
                return carry

            lax.fori_loop(0, PV // 4, aloop, 0)
            pltpu.sync_copy(xb, agg_out.at[cid, slh])
            return carry

        lax.fori_loop(0, NPASS, stage3, 0)

    pl.run_scoped(scoped,
                  pltpu.VMEM((PR, 8), jnp.float32),
                  pltpu.VMEM((PR,), jnp.float32),
                  pltpu.VMEM((PR,), jnp.float32),
                  pltpu.VMEM((PR,), jnp.float32),
                  pltpu.VMEM((CH * ROW,), jnp.int32),
                  pltpu.VMEM((CH * ROW,), jnp.int32),
                  pltpu.VMEM((CH * ROW,), jnp.int32),
                  pltpu.VMEM((CH * ROW,), jnp.int32),
                  pltpu.VMEM((CH * ROW, 8), jnp.float32),
                  pltpu.VMEM((CH * ROW, 8), jnp.float32),
                  pltpu.SemaphoreType.DMA,
                  pltpu.SemaphoreType.DMA,
                  pltpu.SemaphoreType.DMA,
                  pltpu.SemaphoreType.DMA,
                  pltpu.SemaphoreType.DMA,
                  pltpu.SemaphoreType.DMA)


def _hidden_body(a_ref, w_ref, b_ref, hid_ref):
    agg = a_ref[0] + a_ref[1]
    hid_ref[...] = (
        jnp.dot(agg, w_ref[...], preferred_element_type=jnp.float32) + b_ref[...]
    )


def _hidden(agg_parts, W, b2):
    blk = 2000
    grid = (N // blk,)
    return pl.pallas_call(
        _hidden_body,
        grid=grid,
        in_specs=[
            pl.BlockSpec((NC, blk, 8), lambda i: (0, i, 0)),
            pl.BlockSpec((8, 16), lambda i: (0, 0)),
            pl.BlockSpec((1, 16), lambda i: (0, 0)),
        ],
        out_specs=pl.BlockSpec((blk, 16), lambda i: (i, 0)),
        out_shape=jax.ShapeDtypeStruct((N, 16), jnp.float32),
    )(agg_parts, W, b2)


def _lin_body(h_ref, wl_ref, bl_ref, out_ref):
    h = jnp.maximum(h_ref[...], 0.0)
    out_ref[...] = (
        jnp.dot(h, wl_ref[...], preferred_element_type=jnp.float32) + bl_ref[...]
    )


def _linear(h2, Wl, bl2):
    blk = 2000
    grid = (N // 5 // blk,)
    return pl.pallas_call(
        _lin_body,
        grid=grid,
        in_specs=[
            pl.BlockSpec((blk, 80), lambda i: (i, 0)),
            pl.BlockSpec((80, 445), lambda i: (0, 0)),
            pl.BlockSpec((1, 445), lambda i: (0, 0)),
        ],
        out_specs=pl.BlockSpec((blk, 445), lambda i: (i, 0)),
        out_shape=jax.ShapeDtypeStruct((N // 5, 445), jnp.float32),
    )(h2, Wl, bl2)


def kernel(x, edge_index, W, b, Wl, bl):
    src = edge_index[0]
    dst = edge_index[1]
    pad = jnp.full((E_PAD - E,), TRASH, dtype=jnp.int32)
    src_p = jnp.concatenate([src, pad])
    dst_p = jnp.concatenate([dst, pad])
    x_pad = jnp.pad(x, ((0, NP - N), (0, 0)))

    deg_parts = _deg_kernel(dst_p)
    agg_parts = _agg_kernel(src_p, dst_p, x_pad, deg_parts)
    hidden = _hidden(agg_parts, W, b.reshape(1, 16))
    h2 = hidden.reshape(N // 5, 80)
    out = _linear(h2, Wl, bl.reshape(1, 445)).reshape(N // 5, 89, 5)
    return (out, hidden)
